# Initial kernel scaffold; baseline (speedup 1.0000x reference)
import jax
import jax.numpy as jnp
from jax.experimental import pallas as pl


def kernel(inputs, embedding):
    raise NotImplementedError("write your pallas kernel here")


# speedup vs baseline: 2.7522x; 2.7522x over previous
import jax
import jax.numpy as jnp
from jax.experimental import pallas as pl


def _copy_kernel(emb_ref, out_ref):
    out_ref[...] = emb_ref[...]


def kernel(inputs, embedding):
    seq_len, embd_dim = embedding.shape
    block_rows = 512
    grid = (seq_len // block_rows,)
    out = pl.pallas_call(
        _copy_kernel,
        grid=grid,
        in_specs=[pl.BlockSpec((block_rows, embd_dim), lambda i: (i, 0))],
        out_specs=pl.BlockSpec((block_rows, embd_dim), lambda i: (i, 0)),
        out_shape=jax.ShapeDtypeStruct((seq_len, embd_dim), embedding.dtype),
    )(embedding)
    return out[None]


# speedup vs baseline: 3.2148x; 1.1681x over previous
import jax
import jax.numpy as jnp
from jax.experimental import pallas as pl


def _copy_kernel(emb_ref, out_ref):
    out_ref[...] = emb_ref[...]


def kernel(inputs, embedding):
    seq_len, embd_dim = embedding.shape
    block_rows = 1024
    grid = (seq_len // block_rows,)
    out = pl.pallas_call(
        _copy_kernel,
        grid=grid,
        in_specs=[pl.BlockSpec((block_rows, embd_dim), lambda i: (i, 0))],
        out_specs=pl.BlockSpec((block_rows, embd_dim), lambda i: (i, 0)),
        out_shape=jax.ShapeDtypeStruct((seq_len, embd_dim), embedding.dtype),
    )(embedding)
    return out[None]


# speedup vs baseline: 3.4511x; 1.0735x over previous
import jax
import jax.numpy as jnp
from jax.experimental import pallas as pl


def _copy_kernel(emb_ref, out_ref):
    out_ref[...] = emb_ref[...]


def kernel(inputs, embedding):
    seq_len, embd_dim = embedding.shape
    block_rows = 2048
    grid = (seq_len // block_rows,)
    out = pl.pallas_call(
        _copy_kernel,
        grid=grid,
        in_specs=[pl.BlockSpec((block_rows, embd_dim), lambda i: (i, 0))],
        out_specs=pl.BlockSpec((block_rows, embd_dim), lambda i: (i, 0)),
        out_shape=jax.ShapeDtypeStruct((seq_len, embd_dim), embedding.dtype),
    )(embedding)
    return out[None]


# speedup vs baseline: 3.6669x; 1.0625x over previous
import jax
import jax.numpy as jnp
from jax.experimental import pallas as pl


def _copy_kernel(emb_ref, out_ref):
    out_ref[...] = emb_ref[...]


def kernel(inputs, embedding):
    seq_len, embd_dim = embedding.shape
    block_rows = 4096
    grid = (seq_len // block_rows,)
    out = pl.pallas_call(
        _copy_kernel,
        grid=grid,
        in_specs=[pl.BlockSpec((block_rows, embd_dim), lambda i: (i, 0))],
        out_specs=pl.BlockSpec((block_rows, embd_dim), lambda i: (i, 0)),
        out_shape=jax.ShapeDtypeStruct((seq_len, embd_dim), embedding.dtype),
    )(embedding)
    return out[None]
